# baseline (speedup 1.0000x reference)
import jax
import jax.numpy as jnp
from jax import lax
from jax.experimental import pallas as pl
from jax.experimental.pallas import tpu as pltpu
from jax.experimental.pallas import tpu_sc as plsc

D = 64
NC = 2
NS = 16
NW = NC * NS
NBUF = 8
LOOKA = 4


def _emb_body(x_hbm, table_hbm, out_hbm, idx_v, rows_v, gsem, wsem):
    nch = x_hbm.shape[0] // NW
    wid = lax.axis_index("s") * NC + lax.axis_index("c")
    base = wid * nch

    pltpu.sync_copy(x_hbm.at[pl.ds(base, nch)], idx_v)

    def gather_start(c, b):
        pltpu.make_async_copy(
            table_hbm.at[idx_v.at[c]], rows_v.at[b], gsem.at[b]
        ).start()

    def gather_wait(c, b):
        pltpu.make_async_copy(
            table_hbm.at[idx_v.at[c]], rows_v.at[b], gsem.at[b]
        ).wait()

    def write_start(c, b):
        pltpu.make_async_copy(
            rows_v.at[b], out_hbm.at[base + c], wsem.at[b]
        ).start()

    def write_wait(c, b):
        pltpu.make_async_copy(
            rows_v.at[b], out_hbm.at[base + c], wsem.at[b]
        ).wait()

    for c in range(LOOKA):
        gather_start(c, c % NBUF)

    n_groups = nch // NBUF

    def group(g, carry):
        for u in range(NBUF):
            c = g * NBUF + u
            b = u
            gather_wait(c, b)
            write_start(c, b)
            q = c + LOOKA
            bq = (u + LOOKA) % NBUF

            @pl.when(q < nch)
            def _arm_next():
                @pl.when(q >= NBUF)
                def _drain_old_write():
                    write_wait(q - NBUF, bq)

                gather_start(q, bq)

        return carry

    lax.fori_loop(0, n_groups, group, 0, unroll=False)

    for c in range(nch - NBUF, nch):
        write_wait(c, c % NBUF)


def kernel(x, table):
    B, S = x.shape
    nch = B // NW
    idx = x.astype(jnp.int32)

    emb = pl.kernel(
        _emb_body,
        out_type=jax.ShapeDtypeStruct((B, S, D), jnp.float32),
        mesh=plsc.VectorSubcoreMesh(
            core_axis_name="c", subcore_axis_name="s",
            num_cores=NC, num_subcores=NS,
        ),
        scratch_types=[
            pltpu.VMEM((nch, S), jnp.int32),
            pltpu.VMEM((NBUF, S, D), jnp.float32),
            pltpu.SemaphoreType.DMA((NBUF,)),
            pltpu.SemaphoreType.DMA((NBUF,)),
        ],
        compiler_params=pltpu.CompilerParams(use_tc_tiling_on_sc=False),
    )
    return emb(idx, table)


# speedup vs baseline: 1.2176x; 1.2176x over previous
import jax
import jax.numpy as jnp
from jax import lax
from jax.experimental import pallas as pl
from jax.experimental.pallas import tpu as pltpu
from jax.experimental.pallas import tpu_sc as plsc

D = 64
DP = 128
NC = 2
NS = 16
NW = NC * NS
NBUF = 4
LOOKA = 2


def _emb_body(x_hbm, table_hbm, out_hbm, idx_v, rows_v, gsem, wsem):
    nch = x_hbm.shape[0] // NW
    S = x_hbm.shape[1]
    wid = lax.axis_index("s") * NC + lax.axis_index("c")
    base = wid * nch

    pltpu.sync_copy(x_hbm.at[pl.ds(base, nch)], idx_v)

    def gather_start(c, b):
        pltpu.make_async_copy(
            table_hbm.at[idx_v.at[c]], rows_v.at[b], gsem.at[b]
        ).start()

    def gather_wait(c, b):
        pltpu.make_async_copy(
            table_hbm.at[idx_v.at[c]], rows_v.at[b], gsem.at[b]
        ).wait()

    def write_start(c, b):
        pltpu.make_async_copy(
            rows_v.at[b], out_hbm.at[pl.ds((base + c) * S, S)], wsem.at[b]
        ).start()

    def write_wait(c, b):
        pltpu.make_async_copy(
            rows_v.at[b], out_hbm.at[pl.ds((base + c) * S, S)], wsem.at[b]
        ).wait()

    for c in range(LOOKA):
        gather_start(c, c % NBUF)

    n_groups = nch // NBUF

    def group(g, carry):
        for u in range(NBUF):
            c = g * NBUF + u
            b = u
            gather_wait(c, b)
            write_start(c, b)
            q = c + LOOKA
            bq = (u + LOOKA) % NBUF

            @pl.when(q < nch)
            def _arm_next():
                @pl.when(q >= NBUF)
                def _drain_old_write():
                    write_wait(q - NBUF, bq)

                gather_start(q, bq)

        return carry

    lax.fori_loop(0, n_groups, group, 0, unroll=False)

    for c in range(nch - NBUF, nch):
        write_wait(c, c % NBUF)


def kernel(x, table):
    B, S = x.shape
    nch = B // NW
    idx = x.astype(jnp.int32)
    tablep = jnp.pad(table, ((0, 0), (0, DP - D)))

    emb = pl.kernel(
        _emb_body,
        out_type=jax.ShapeDtypeStruct((B * S, DP), jnp.float32),
        mesh=plsc.VectorSubcoreMesh(
            core_axis_name="c", subcore_axis_name="s",
            num_cores=NC, num_subcores=NS,
        ),
        scratch_types=[
            pltpu.VMEM((nch, S), jnp.int32),
            pltpu.VMEM((NBUF, S, DP), jnp.float32),
            pltpu.SemaphoreType.DMA((NBUF,)),
            pltpu.SemaphoreType.DMA((NBUF,)),
        ],
        compiler_params=pltpu.CompilerParams(use_tc_tiling_on_sc=False),
    )
    outp = emb(idx, tablep)
    return outp[:, :D].reshape(B, S, D)


# speedup vs baseline: 1.4251x; 1.1704x over previous
import jax
import jax.numpy as jnp
from jax import lax
from jax.experimental import pallas as pl
from jax.experimental.pallas import tpu as pltpu
from jax.experimental.pallas import tpu_sc as plsc

D = 64
DP = 128
NC = 2
NS = 16
NW = NC * NS
NBUF = 4
LOOKA = 2


def _emb_body(x_hbm, table_hbm, out_hbm, idx_v, rows_v, gsem, wsem):
    nch = x_hbm.shape[0] // NW
    S = x_hbm.shape[1]
    wid = lax.axis_index("s") * NC + lax.axis_index("c")
    base = wid * nch

    pltpu.sync_copy(x_hbm.at[pl.ds(base, nch)], idx_v)

    def gather_start(c, b):
        pltpu.make_async_copy(
            table_hbm.at[idx_v.at[c]], rows_v.at[b], gsem.at[b]
        ).start()

    def gather_wait(c, b):
        pltpu.make_async_copy(
            table_hbm.at[idx_v.at[c]], rows_v.at[b], gsem.at[b]
        ).wait()

    def write_start(c, b):
        pltpu.make_async_copy(
            rows_v.at[b],
            out_hbm.at[pl.ds((base + c) * S, S), pl.ds(0, D)],
            wsem.at[b],
        ).start()

    def write_wait(c, b):
        pltpu.make_async_copy(
            rows_v.at[b],
            out_hbm.at[pl.ds((base + c) * S, S), pl.ds(0, D)],
            wsem.at[b],
        ).wait()

    for c in range(LOOKA):
        gather_start(c, c % NBUF)

    n_groups = nch // NBUF

    def group(g, carry):
        for u in range(NBUF):
            c = g * NBUF + u
            b = u
            gather_wait(c, b)
            write_start(c, b)
            q = c + LOOKA
            bq = (u + LOOKA) % NBUF

            @pl.when(q < nch)
            def _arm_next():
                @pl.when(q >= NBUF)
                def _drain_old_write():
                    write_wait(q - NBUF, bq)

                gather_start(q, bq)

        return carry

    lax.fori_loop(0, n_groups, group, 0, unroll=False)

    for c in range(nch - NBUF, nch):
        write_wait(c, c % NBUF)


def kernel(x, table):
    B, S = x.shape
    nch = B // NW
    idx = x.astype(jnp.int32) * 2
    tablep = jnp.pad(table, ((0, 0), (0, DP - D)))
    tablep2 = tablep.reshape(2 * tablep.shape[0], D)

    emb = pl.kernel(
        _emb_body,
        out_type=jax.ShapeDtypeStruct((B * S, DP), jnp.float32),
        mesh=plsc.VectorSubcoreMesh(
            core_axis_name="c", subcore_axis_name="s",
            num_cores=NC, num_subcores=NS,
        ),
        scratch_types=[
            pltpu.VMEM((nch, S), jnp.int32),
            pltpu.VMEM((NBUF, S, D), jnp.float32),
            pltpu.SemaphoreType.DMA((NBUF,)),
            pltpu.SemaphoreType.DMA((NBUF,)),
        ],
        compiler_params=pltpu.CompilerParams(use_tc_tiling_on_sc=False),
    )
    outp = emb(idx, tablep2)
    return outp[:, :D].reshape(B, S, D)


# speedup vs baseline: 1.4256x; 1.0003x over previous
import jax
import jax.numpy as jnp
from jax import lax
from jax.experimental import pallas as pl
from jax.experimental.pallas import tpu as pltpu
from jax.experimental.pallas import tpu_sc as plsc

D = 64
DP = 128
NC = 2
NS = 16
NW = NC * NS
NBUF = 8
LOOKA = 5


def _emb_body(x_hbm, table_hbm, out_hbm, idx_v, rows_v, gsem, wsem):
    nch = x_hbm.shape[0] // NW
    S = x_hbm.shape[1]
    wid = lax.axis_index("s") * NC + lax.axis_index("c")
    base = wid * nch

    pltpu.sync_copy(x_hbm.at[pl.ds(base, nch)], idx_v)

    def gather_start(c, b):
        pltpu.make_async_copy(
            table_hbm.at[idx_v.at[c]], rows_v.at[b], gsem.at[b]
        ).start()

    def gather_wait(c, b):
        pltpu.make_async_copy(
            table_hbm.at[idx_v.at[c]], rows_v.at[b], gsem.at[b]
        ).wait()

    def write_start(c, b):
        pltpu.make_async_copy(
            rows_v.at[b],
            out_hbm.at[pl.ds((base + c) * S, S), pl.ds(0, D)],
            wsem.at[b],
        ).start()

    def write_wait(c, b):
        pltpu.make_async_copy(
            rows_v.at[b],
            out_hbm.at[pl.ds((base + c) * S, S), pl.ds(0, D)],
            wsem.at[b],
        ).wait()

    for c in range(LOOKA):
        gather_start(c, c % NBUF)

    n_groups = nch // NBUF

    def group(g, carry):
        for u in range(NBUF):
            c = g * NBUF + u
            b = u
            gather_wait(c, b)
            write_start(c, b)
            q = c + LOOKA
            bq = (u + LOOKA) % NBUF

            @pl.when(q < nch)
            def _arm_next():
                @pl.when(q >= NBUF)
                def _drain_old_write():
                    write_wait(q - NBUF, bq)

                gather_start(q, bq)

        return carry

    lax.fori_loop(0, n_groups, group, 0, unroll=False)

    for c in range(nch - NBUF, nch):
        write_wait(c, c % NBUF)


def kernel(x, table):
    B, S = x.shape
    nch = B // NW
    idx = x.astype(jnp.int32) * 2
    tablep = jnp.pad(table, ((0, 0), (0, DP - D)))
    tablep2 = tablep.reshape(2 * tablep.shape[0], D)

    emb = pl.kernel(
        _emb_body,
        out_type=jax.ShapeDtypeStruct((B * S, DP), jnp.float32),
        mesh=plsc.VectorSubcoreMesh(
            core_axis_name="c", subcore_axis_name="s",
            num_cores=NC, num_subcores=NS,
        ),
        scratch_types=[
            pltpu.VMEM((nch, S), jnp.int32),
            pltpu.VMEM((NBUF, S, D), jnp.float32),
            pltpu.SemaphoreType.DMA((NBUF,)),
            pltpu.SemaphoreType.DMA((NBUF,)),
        ],
        compiler_params=pltpu.CompilerParams(use_tc_tiling_on_sc=False),
    )
    outp = emb(idx, tablep2)
    return outp[:, :D].reshape(B, S, D)
